# Initial kernel scaffold; baseline (speedup 1.0000x reference)
import jax
import jax.numpy as jnp
from jax.experimental import pallas as pl


def kernel(feats, batch, W, b):
    raise NotImplementedError("write your pallas kernel here")


# speedup vs baseline: 1.6791x; 1.6791x over previous
import functools

import jax
import jax.numpy as jnp
from jax import lax
from jax.experimental import pallas as pl
from jax.experimental.pallas import tpu as pltpu
from jax.experimental.pallas import tpu_sc as plsc

N_ROWS = 160000
D = 256
NSEG = 10000
L = 16
DL = D // L
NW = 32
SPW = 313
NSEG_PAD = NW * SPW
CHUNK = 64
CHUNK_WORDS = CHUNK * D
ROWB_PAD = 48


def _sc_body(feats_hbm, ids_hbm, rowb_hbm, wb_hbm, out_hbm,
             wb_v, rowb_v, ids_v, feats_v, acc_v):
    cid = lax.axis_index("c")
    sid = lax.axis_index("s")
    wid = sid * 2 + cid

    pltpu.sync_copy(rowb_hbm, rowb_v)
    pltpu.sync_copy(wb_hbm, wb_v)

    iota = lax.iota(jnp.int32, L)

    def lane_scalar(vec, lane):
        return jnp.max(jnp.where(iota == lane, vec, -1))

    def rowb_at(idx):
        v0 = rowb_v[pl.ds(0, L)]
        v1 = rowb_v[pl.ds(L, L)]
        v2 = rowb_v[pl.ds(2 * L, L)]
        lo = jnp.where(idx < L, lane_scalar(v0, idx), lane_scalar(v1, idx - L))
        return jnp.where(idx < 2 * L, lo, lane_scalar(v2, idx - 2 * L))

    r_lo = rowb_at(wid)
    r_hi = rowb_at(wid + 1)
    seg_base = wid * SPW

    Ws = [wb_v[pl.ds(j * L, L)] for j in range(DL)]
    bsplat = wb_v[pl.ds(D, L)]

    zero = jnp.zeros((L,), jnp.float32)

    def zrow(r, carry):
        base = r * D
        for j in range(DL):
            acc_v[pl.ds(base + j * L, L)] = zero
        return carry

    lax.fori_loop(0, SPW, zrow, 0)

    k_lo = r_lo // CHUNK
    k_hi = jnp.maximum(lax.div(r_hi + CHUNK - 1, CHUNK), k_lo)

    def chunk_body(k, carry):
        row0 = k * CHUNK
        pltpu.sync_copy(feats_hbm.at[pl.ds(row0 * D, CHUNK_WORDS)], feats_v)
        pltpu.sync_copy(ids_hbm.at[pl.ds(row0, CHUNK)], ids_v)
        i_lo = jnp.maximum(r_lo - row0, 0)
        i_hi = jnp.minimum(r_hi - row0, CHUNK)

        def row_body(i, c2):
            idv = ids_v[pl.ds((i // L) * L, L)]
            bid = lane_scalar(idv, i % L)
            sloc = bid - seg_base
            fbase = i * D
            f = [feats_v[pl.ds(fbase + j * L, L)] for j in range(DL)]
            dot = f[0] * Ws[0]
            for j in range(1, DL):
                dot = dot + f[j] * Ws[j]
            s = jnp.sum(dot)
            sv = jnp.full((L,), s, jnp.float32) + bsplat
            wv = 1.0 / (1.0 + jnp.exp(-sv))
            abase = sloc * D
            for j in range(DL):
                plsc.addupdate(acc_v.at[pl.ds(abase + j * L, L)], wv * f[j])
            return c2

        lax.fori_loop(i_lo, i_hi, row_body, 0)
        return carry

    lax.fori_loop(k_lo, k_hi, chunk_body, 0)

    pltpu.sync_copy(acc_v, out_hbm.at[pl.ds(seg_base * D, SPW * D)])


_sc_call = functools.partial(
    pl.kernel,
    mesh=plsc.VectorSubcoreMesh(core_axis_name="c", subcore_axis_name="s"),
    compiler_params=pltpu.CompilerParams(needs_layout_passes=False),
    out_type=jax.ShapeDtypeStruct((NSEG_PAD * D,), jnp.float32),
    scratch_types=[
        pltpu.VMEM((D + L,), jnp.float32),
        pltpu.VMEM((ROWB_PAD,), jnp.int32),
        pltpu.VMEM((CHUNK,), jnp.int32),
        pltpu.VMEM((CHUNK_WORDS,), jnp.float32),
        pltpu.VMEM((SPW * D,), jnp.float32),
    ],
)(_sc_body)


def kernel(feats, batch, W, b):
    feats_flat = feats.reshape(-1)
    seg_bounds = jnp.arange(NW + 1, dtype=jnp.int32) * SPW
    rowb = jnp.searchsorted(batch, seg_bounds, side="left").astype(jnp.int32)
    rowb = jnp.concatenate(
        [rowb, jnp.full((ROWB_PAD - NW - 1,), N_ROWS, jnp.int32)])
    wb = jnp.concatenate([W[:, 0], jnp.full((L,), b[0], jnp.float32)])
    out_flat = _sc_call(feats_flat, batch, rowb, wb)
    return out_flat[: NSEG * D].reshape(NSEG, D)


# speedup vs baseline: 2.1048x; 1.2536x over previous
import functools

import jax
import jax.numpy as jnp
from jax import lax
from jax.experimental import pallas as pl
from jax.experimental.pallas import tpu as pltpu
from jax.experimental.pallas import tpu_sc as plsc

N_ROWS = 160000
D = 256
NSEG = 10000
L = 16
DL = D // L
NW = 32
SPW = 313
NSEG_PAD = NW * SPW
CHUNK = 80
CHUNK_WORDS = CHUNK * D
MAXK = N_ROWS // CHUNK - 1
ROWB_PAD = 48


def _sc_body(feats_hbm, ids_hbm, rowb_hbm, wb_hbm, out_hbm,
             wb_v, rowb_v, ids0_v, ids1_v, f0_v, f1_v, acc_v,
             sf0, sf1, si0, si1):
    cid = lax.axis_index("c")
    sid = lax.axis_index("s")
    wid = sid * 2 + cid

    pltpu.sync_copy(rowb_hbm, rowb_v)
    pltpu.sync_copy(wb_hbm, wb_v)

    r_lo = rowb_v[pl.ds(wid, L)][0]
    r_hi = rowb_v[pl.ds(wid + 1, L)][0]
    seg_base = wid * SPW

    Ws = [wb_v[pl.ds(j * L, L)] for j in range(DL)]
    bsplat = wb_v[pl.ds(D, L)]

    zero = jnp.zeros((L,), jnp.float32)

    def zrow(r, carry):
        base = r * D
        for j in range(DL):
            acc_v[pl.ds(base + j * L, L)] = zero
        return carry

    lax.fori_loop(0, SPW, zrow, 0)

    def handles(k, f_v, i_v, s_f, s_i):
        kc = jnp.minimum(k, MAXK)
        row0 = kc * CHUNK
        h_f = pltpu.make_async_copy(
            feats_hbm.at[pl.ds(row0 * D, CHUNK_WORDS)], f_v, s_f)
        h_i = pltpu.make_async_copy(
            ids_hbm.at[pl.ds(row0, CHUNK)], i_v.at[pl.ds(0, CHUNK)], s_i)
        return h_f, h_i

    def start(k, f_v, i_v, s_f, s_i):
        h_f, h_i = handles(k, f_v, i_v, s_f, s_i)
        h_f.start()
        h_i.start()

    def wait(k, f_v, i_v, s_f, s_i):
        h_f, h_i = handles(k, f_v, i_v, s_f, s_i)
        h_f.wait()
        h_i.wait()

    def process(k, i_v, f_v):
        row0 = k * CHUNK

        def row_body(i, carry):
            g = row0 + i
            bid = i_v[pl.ds(i, L)][0]
            sloc = jnp.minimum(jnp.maximum(bid - seg_base, 0), SPW - 1)
            fbase = i * D
            f = [f_v[pl.ds(fbase + j * L, L)] for j in range(DL)]
            dot = f[0] * Ws[0]
            for j in range(1, DL):
                dot = dot + f[j] * Ws[j]
            sv = jnp.full((L,), jnp.sum(dot), jnp.float32) + bsplat
            wv = 1.0 / (1.0 + jnp.exp(-sv))
            act = jnp.where((g >= r_lo) & (g < r_hi), 1.0, 0.0)
            wv = wv * act
            abase = sloc * D
            for j in range(DL):
                plsc.addupdate(acc_v.at[pl.ds(abase + j * L, L)], wv * f[j])
            return carry

        lax.fori_loop(0, CHUNK, row_body, 0)

    k_lo = r_lo // CHUNK
    k_hi = jnp.maximum(lax.div(r_hi + CHUNK - 1, CHUNK), k_lo)
    npairs = (k_hi - k_lo + 1) // 2

    start(k_lo, f0_v, ids0_v, sf0, si0)

    def pair_body(p, carry):
        k0 = k_lo + 2 * p
        start(k0 + 1, f1_v, ids1_v, sf1, si1)
        wait(k0, f0_v, ids0_v, sf0, si0)
        process(k0, ids0_v, f0_v)
        start(k0 + 2, f0_v, ids0_v, sf0, si0)
        wait(k0 + 1, f1_v, ids1_v, sf1, si1)
        process(k0 + 1, ids1_v, f1_v)
        return carry

    lax.fori_loop(0, npairs, pair_body, 0)

    wait(k_lo, f0_v, ids0_v, sf0, si0)

    pltpu.sync_copy(acc_v, out_hbm.at[pl.ds(seg_base * D, SPW * D)])


_sc_call = functools.partial(
    pl.kernel,
    mesh=plsc.VectorSubcoreMesh(core_axis_name="c", subcore_axis_name="s"),
    compiler_params=pltpu.CompilerParams(needs_layout_passes=False),
    out_type=jax.ShapeDtypeStruct((NSEG_PAD * D,), jnp.float32),
    scratch_types=[
        pltpu.VMEM((D + L,), jnp.float32),
        pltpu.VMEM((ROWB_PAD,), jnp.int32),
        pltpu.VMEM((CHUNK + L,), jnp.int32),
        pltpu.VMEM((CHUNK + L,), jnp.int32),
        pltpu.VMEM((CHUNK_WORDS,), jnp.float32),
        pltpu.VMEM((CHUNK_WORDS,), jnp.float32),
        pltpu.VMEM((SPW * D,), jnp.float32),
        pltpu.SemaphoreType.DMA,
        pltpu.SemaphoreType.DMA,
        pltpu.SemaphoreType.DMA,
        pltpu.SemaphoreType.DMA,
    ],
)(_sc_body)


def kernel(feats, batch, W, b):
    feats_flat = feats.reshape(-1)
    seg_bounds = jnp.arange(NW + 1, dtype=jnp.int32) * SPW
    rowb = jnp.searchsorted(batch, seg_bounds, side="left").astype(jnp.int32)
    rowb = jnp.concatenate(
        [rowb, jnp.full((ROWB_PAD - NW - 1,), N_ROWS, jnp.int32)])
    wb = jnp.concatenate([W[:, 0], jnp.full((L,), b[0], jnp.float32)])
    out_flat = _sc_call(feats_flat, batch, rowb, wb)
    return out_flat[: NSEG * D].reshape(NSEG, D)


# speedup vs baseline: 2.9725x; 1.4122x over previous
import functools

import jax
import jax.numpy as jnp
from jax import lax
from jax.experimental import pallas as pl
from jax.experimental.pallas import tpu as pltpu
from jax.experimental.pallas import tpu_sc as plsc

N_ROWS = 160000
D = 256
NSEG = 10000
L = 16
DL = D // L
NW = 32
SPW = 313
NSEG_PAD = NW * SPW
CHUNK = 80
CHUNK_WORDS = CHUNK * D
MAXK = N_ROWS // CHUNK - 1
ROWB_PAD = 48


def _sc_body(feats_hbm, ids_hbm, rowb_hbm, wb_hbm, out_hbm,
             wb_v, rowb_v, ids0_v, ids1_v, f0_v, f1_v, acc_v,
             sf0, sf1, si0, si1):
    cid = lax.axis_index("c")
    sid = lax.axis_index("s")
    wid = sid * 2 + cid

    pltpu.sync_copy(rowb_hbm, rowb_v)
    pltpu.sync_copy(wb_hbm, wb_v)

    r_lo = rowb_v[pl.ds(wid, L)][0]
    r_hi = rowb_v[pl.ds(wid + 1, L)][0]
    seg_base = wid * SPW

    Ws = [wb_v[pl.ds(j * L, L)] for j in range(DL)]
    bsplat = wb_v[pl.ds(D, L)]

    zero = jnp.zeros((L,), jnp.float32)

    def zrow(r, carry):
        base = r * D
        for j in range(DL):
            acc_v[pl.ds(base + j * L, L)] = zero
        return carry

    lax.fori_loop(0, SPW, zrow, 0)

    def handles(k, f_v, i_v, s_f, s_i):
        kc = jnp.minimum(k, MAXK)
        row0 = kc * CHUNK
        h_f = pltpu.make_async_copy(
            feats_hbm.at[pl.ds(row0 * D, CHUNK_WORDS)], f_v, s_f)
        h_i = pltpu.make_async_copy(
            ids_hbm.at[pl.ds(row0, CHUNK)], i_v.at[pl.ds(0, CHUNK)], s_i)
        return h_f, h_i

    def start(k, f_v, i_v, s_f, s_i):
        h_f, h_i = handles(k, f_v, i_v, s_f, s_i)
        h_f.start()
        h_i.start()

    def wait(k, f_v, i_v, s_f, s_i):
        h_f, h_i = handles(k, f_v, i_v, s_f, s_i)
        h_f.wait()
        h_i.wait()

    def process(k, i_v, f_v):
        row0 = k * CHUNK

        def one_row(i):
            g = row0 + i
            bid = i_v[pl.ds(i, L)][0]
            sloc = jnp.minimum(jnp.maximum(bid - seg_base, 0), SPW - 1)
            fbase = i * D
            f = [f_v[pl.ds(fbase + j * L, L)] for j in range(DL)]
            t = [f[j] * Ws[j] for j in range(DL)]
            while len(t) > 1:
                t = [t[2 * j] + t[2 * j + 1] for j in range(len(t) // 2)]
            sv = jnp.full((L,), jnp.sum(t[0]), jnp.float32) + bsplat
            wv = 1.0 / (1.0 + jnp.exp(-sv))
            act = jnp.where((g >= r_lo) & (g < r_hi), 1.0, 0.0)
            wv = wv * act
            return sloc * D, [wv * f[j] for j in range(DL)]

        def pair_rows(p, carry):
            a_base, a_vals = one_row(2 * p)
            b_base, b_vals = one_row(2 * p + 1)
            for j in range(DL):
                plsc.addupdate(acc_v.at[pl.ds(a_base + j * L, L)], a_vals[j])
            for j in range(DL):
                plsc.addupdate(acc_v.at[pl.ds(b_base + j * L, L)], b_vals[j])
            return carry

        lax.fori_loop(0, CHUNK // 2, pair_rows, 0)

    k_lo = r_lo // CHUNK
    k_hi = jnp.maximum(lax.div(r_hi + CHUNK - 1, CHUNK), k_lo)
    npairs = (k_hi - k_lo + 1) // 2

    start(k_lo, f0_v, ids0_v, sf0, si0)

    def pair_body(p, carry):
        k0 = k_lo + 2 * p
        start(k0 + 1, f1_v, ids1_v, sf1, si1)
        wait(k0, f0_v, ids0_v, sf0, si0)
        process(k0, ids0_v, f0_v)
        start(k0 + 2, f0_v, ids0_v, sf0, si0)
        wait(k0 + 1, f1_v, ids1_v, sf1, si1)
        process(k0 + 1, ids1_v, f1_v)
        return carry

    lax.fori_loop(0, npairs, pair_body, 0)

    wait(k_lo, f0_v, ids0_v, sf0, si0)

    pltpu.sync_copy(acc_v, out_hbm.at[pl.ds(seg_base * D, SPW * D)])


_sc_call = functools.partial(
    pl.kernel,
    mesh=plsc.VectorSubcoreMesh(core_axis_name="c", subcore_axis_name="s"),
    compiler_params=pltpu.CompilerParams(needs_layout_passes=False),
    out_type=jax.ShapeDtypeStruct((NSEG_PAD * D,), jnp.float32),
    scratch_types=[
        pltpu.VMEM((D + L,), jnp.float32),
        pltpu.VMEM((ROWB_PAD,), jnp.int32),
        pltpu.VMEM((CHUNK + L,), jnp.int32),
        pltpu.VMEM((CHUNK + L,), jnp.int32),
        pltpu.VMEM((CHUNK_WORDS,), jnp.float32),
        pltpu.VMEM((CHUNK_WORDS,), jnp.float32),
        pltpu.VMEM((SPW * D,), jnp.float32),
        pltpu.SemaphoreType.DMA,
        pltpu.SemaphoreType.DMA,
        pltpu.SemaphoreType.DMA,
        pltpu.SemaphoreType.DMA,
    ],
)(_sc_body)


def kernel(feats, batch, W, b):
    feats_flat = feats.reshape(-1)
    seg_bounds = jnp.arange(NW + 1, dtype=jnp.int32) * SPW
    rowb = jnp.searchsorted(batch, seg_bounds, side="left").astype(jnp.int32)
    rowb = jnp.concatenate(
        [rowb, jnp.full((ROWB_PAD - NW - 1,), N_ROWS, jnp.int32)])
    wb = jnp.concatenate([W[:, 0], jnp.full((L,), b[0], jnp.float32)])
    out_flat = _sc_call(feats_flat, batch, rowb, wb)
    return out_flat[: NSEG * D].reshape(NSEG, D)
